# baseline (speedup 1.0000x reference)
import jax
import jax.numpy as jnp
from jax.experimental import pallas as pl
from jax.experimental.pallas import tpu as pltpu

_BF16 = jnp.bfloat16


def _conv_pool_body(q_ref, w_ref, b_ref, o_ref):
    w = w_ref[...]
    a0 = jnp.dot(q_ref[0], w, preferred_element_type=jnp.float32)
    a1 = jnp.dot(q_ref[1], w, preferred_element_type=jnp.float32)
    a2 = jnp.dot(q_ref[2], w, preferred_element_type=jnp.float32)
    a3 = jnp.dot(q_ref[3], w, preferred_element_type=jnp.float32)
    m = jnp.maximum(jnp.maximum(a0, a1), jnp.maximum(a2, a3))
    o_ref[...] = jnp.maximum(m + b_ref[...], 0.0).astype(o_ref.dtype)


def _conv_layer(q, w_mat, shift, tile_r, out_dtype):
    _, rows, k = q.shape
    c_out = w_mat.shape[1]
    return pl.pallas_call(
        _conv_pool_body,
        out_shape=jax.ShapeDtypeStruct((rows, c_out), out_dtype),
        grid=(pl.cdiv(rows, tile_r),),
        in_specs=[
            pl.BlockSpec((4, tile_r, k), lambda r: (0, r, 0)),
            pl.BlockSpec((k, c_out), lambda r: (0, 0)),
            pl.BlockSpec((1, c_out), lambda r: (0, 0)),
        ],
        out_specs=pl.BlockSpec((tile_r, c_out), lambda r: (r, 0)),
        compiler_params=pltpu.CompilerParams(
            dimension_semantics=("parallel",)),
    )(q, w_mat.astype(q.dtype), shift)


def _fc_head_body(x_ref, w1_ref, b1_ref, w2_ref, b2_ref, o_ref):
    h = jnp.dot(x_ref[...], w1_ref[...],
                preferred_element_type=jnp.float32) + b1_ref[...]
    logits = jnp.dot(h, w2_ref[...],
                     preferred_element_type=jnp.float32) + b2_ref[...]
    mx = jnp.max(logits, axis=-1, keepdims=True)
    e = jnp.exp(logits - mx)
    o_ref[...] = e / jnp.sum(e, axis=-1, keepdims=True)


def _fc_head(feat, w1, b1, w2, b2):
    n, f = feat.shape
    h = w1.shape[1]
    o = w2.shape[1]
    return pl.pallas_call(
        _fc_head_body,
        out_shape=jax.ShapeDtypeStruct((n, o), jnp.float32),
        grid=(1,),
        in_specs=[
            pl.BlockSpec((n, f), lambda i: (0, 0)),
            pl.BlockSpec((f, h), lambda i: (0, 0)),
            pl.BlockSpec((1, h), lambda i: (0, 0)),
            pl.BlockSpec((h, o), lambda i: (0, 0)),
            pl.BlockSpec((1, o), lambda i: (0, 0)),
        ],
        out_specs=pl.BlockSpec((n, o), lambda i: (0, 0)),
    )(feat, w1, b1, w2, b2)


def _quad_patches(x, hp, wp):
    n, _, _, c = x.shape
    quads = []
    for dh in (0, 1):
        for dw in (0, 1):
            taps = []
            for kh in range(3):
                for kw in range(3):
                    r0 = 2 * dh + kh
                    c0 = 2 * dw + kw
                    taps.append(x[:, r0:r0 + 4 * hp:4, c0:c0 + 4 * wp:4, :])
            quads.append(jnp.concatenate(taps, axis=-1))
    q = jnp.stack(quads, axis=0)
    return q.reshape(4, n * hp * wp, 9 * c)


def kernel(x, w_mat0, shift0, w_mat1, shift1, w_mat2, shift2, w1, b1, w2, b2):
    n = x.shape[0]
    xh = jnp.transpose(x, (0, 2, 3, 1)).astype(_BF16)

    q0 = _quad_patches(xh, 93, 76)
    y0 = _conv_layer(q0, w_mat0, shift0, 8192, _BF16)

    q1 = _quad_patches(y0.reshape(n, 93, 76, 10), 23, 18)
    y1 = _conv_layer(q1, w_mat1, shift1, 6624, _BF16)

    q2 = _quad_patches(y1.reshape(n, 23, 18, 20), 5, 4)
    y2 = _conv_layer(q2, w_mat2, shift2, n * 20, jnp.float32)

    feat = y2.reshape(n, 800)
    w1p = w1.reshape(40, 20, 100).transpose(1, 0, 2).reshape(800, 100)
    return _fc_head(feat, w1p, b1, w2, b2)


# speedup vs baseline: 5.2840x; 5.2840x over previous
import jax
import jax.numpy as jnp
from jax.experimental import pallas as pl
from jax.experimental.pallas import tpu as pltpu

_BF16 = jnp.bfloat16


def _phase_pack_nchw(x, r, j):
    n, c, h, w = x.shape
    xp = jnp.pad(x, ((0, 0), (0, 0), (0, 4 * r - h), (0, 4 * j - w)))
    xp = xp.reshape(n, c, r, 4, j, 4).transpose(0, 2, 4, 3, 5, 1)
    return xp.reshape(n, r, j, 16 * c).astype(_BF16)


def _phase_pack_nhwc(x, r, j):
    n, h, w, c = x.shape
    xp = jnp.pad(x, ((0, 0), (0, 4 * r - h), (0, 4 * j - w), (0, 0)))
    xp = xp.reshape(n, r, 4, j, 4, c).transpose(0, 1, 3, 2, 4, 5)
    return xp.reshape(n, r, j, 16 * c)


def _make_conv_body(c_in, rout, wd):
    def body(xp_ref, w_ref, b_ref, o_ref):
        xv = xp_ref[...]
        w = w_ref[...]
        acc = None
        for dh in (0, 1):
            for dw in (0, 1):
                pieces = []
                for kh in range(3):
                    for kw in range(3):
                        o = 2 * dh + kh
                        p = 2 * dw + kw
                        q = (o % 4) * 4 + (p % 4)
                        sh = o // 4
                        sw = p // 4
                        pieces.append(
                            xv[sh:sh + rout, sw:sw + wd,
                               q * c_in:(q + 1) * c_in])
                patch = jnp.concatenate(pieces, axis=-1)
                a = jnp.dot(patch.reshape(rout * wd, 9 * c_in), w,
                            preferred_element_type=jnp.float32)
                acc = a if acc is None else jnp.maximum(acc, a)
        o_ref[...] = jnp.maximum(acc + b_ref[...], 0.0).astype(o_ref.dtype)
    return body


def _conv_layer(xp, w_mat, shift, c_in, rout, wd, out_dtype):
    n, r, j, _ = xp.shape
    c_out = w_mat.shape[1]
    return pl.pallas_call(
        _make_conv_body(c_in, rout, wd),
        out_shape=jax.ShapeDtypeStruct((n, rout * wd, c_out), out_dtype),
        grid=(n,),
        in_specs=[
            pl.BlockSpec((None, r, j, 16 * c_in), lambda i: (i, 0, 0, 0)),
            pl.BlockSpec((9 * c_in, c_out), lambda i: (0, 0)),
            pl.BlockSpec((1, c_out), lambda i: (0, 0)),
        ],
        out_specs=pl.BlockSpec((None, rout * wd, c_out), lambda i: (i, 0, 0)),
        compiler_params=pltpu.CompilerParams(
            dimension_semantics=("parallel",)),
    )(xp, w_mat.astype(_BF16), shift)


def _fc_head_body(x_ref, w1_ref, b1_ref, w2_ref, b2_ref, o_ref):
    h = jnp.dot(x_ref[...], w1_ref[...],
                preferred_element_type=jnp.float32) + b1_ref[...]
    logits = jnp.dot(h, w2_ref[...],
                     preferred_element_type=jnp.float32) + b2_ref[...]
    mx = jnp.max(logits, axis=-1, keepdims=True)
    e = jnp.exp(logits - mx)
    o_ref[...] = e / jnp.sum(e, axis=-1, keepdims=True)


def _fc_head(feat, w1, b1, w2, b2):
    n, f = feat.shape
    h = w1.shape[1]
    o = w2.shape[1]
    return pl.pallas_call(
        _fc_head_body,
        out_shape=jax.ShapeDtypeStruct((n, o), jnp.float32),
        grid=(1,),
        in_specs=[
            pl.BlockSpec((n, f), lambda i: (0, 0)),
            pl.BlockSpec((f, h), lambda i: (0, 0)),
            pl.BlockSpec((1, h), lambda i: (0, 0)),
            pl.BlockSpec((h, o), lambda i: (0, 0)),
            pl.BlockSpec((1, o), lambda i: (0, 0)),
        ],
        out_specs=pl.BlockSpec((n, o), lambda i: (0, 0)),
    )(feat, w1, b1, w2, b2)


def kernel(x, w_mat0, shift0, w_mat1, shift1, w_mat2, shift2, w1, b1, w2, b2):
    n = x.shape[0]

    xp0 = _phase_pack_nchw(x, 94, 81)
    y0 = _conv_layer(xp0, w_mat0, shift0, 6, 93, 80, _BF16)
    x1 = y0.reshape(n, 93, 80, 10)[:, :, :76, :]

    xp1 = _phase_pack_nhwc(x1, 24, 33)
    y1 = _conv_layer(xp1, w_mat1, shift1, 10, 23, 32, _BF16)
    x2 = y1.reshape(n, 23, 32, 20)[:, :, :18, :]

    xp2 = _phase_pack_nhwc(x2, 6, 17)
    y2 = _conv_layer(xp2, w_mat2, shift2, 20, 5, 16, jnp.float32)
    feat = y2.reshape(n, 5, 16, 40)[:, :, :4, :].reshape(n, 800)

    w1p = w1.reshape(40, 20, 100).transpose(1, 0, 2).reshape(800, 100)
    return _fc_head(feat, w1p, b1, w2, b2)


# speedup vs baseline: 17.5717x; 3.3254x over previous
import jax
import jax.numpy as jnp
from jax.experimental import pallas as pl
from jax.experimental.pallas import tpu as pltpu

_BF16 = jnp.bfloat16


def _make_conv_body(prep, c_in, rout, wd):
    def body(*args):
        x_refs, w_ref, b_ref, o_ref = args[:-3], args[-3], args[-2], args[-1]
        phases = prep(x_refs)
        w = w_ref[...]
        acc = None
        for dh in (0, 1):
            for dw in (0, 1):
                taps = []
                for kh in range(3):
                    for kw in range(3):
                        o = 2 * dh + kh
                        p = 2 * dw + kw
                        q = (o % 4) * 4 + (p % 4)
                        sh = o // 4
                        sw = p // 4
                        taps.append(
                            phases[q][sh:sh + rout, sw:sw + wd, :])
                patch = jnp.concatenate(taps, axis=-1)
                a = jnp.dot(patch.reshape(rout * wd, 9 * c_in), w,
                            preferred_element_type=jnp.float32)
                acc = a if acc is None else jnp.maximum(acc, a)
        o_ref[...] = jnp.maximum(acc + b_ref[...], 0.0).astype(o_ref.dtype)
    return body


def _conv_layer(xs, in_specs, prep, w_mat, shift, c_in, rout, wd, out_dtype):
    n = xs[0].shape[0]
    c_out = w_mat.shape[1]
    return pl.pallas_call(
        _make_conv_body(prep, c_in, rout, wd),
        out_shape=jax.ShapeDtypeStruct((n, rout * wd, c_out), out_dtype),
        grid=(n,),
        in_specs=in_specs + [
            pl.BlockSpec((9 * c_in, c_out), lambda i: (0, 0)),
            pl.BlockSpec((1, c_out), lambda i: (0, 0)),
        ],
        out_specs=pl.BlockSpec((None, rout * wd, c_out), lambda i: (i, 0, 0)),
        compiler_params=pltpu.CompilerParams(
            dimension_semantics=("parallel",)),
    )(*xs, w_mat.astype(_BF16), shift)


def _fc_head_body(x_ref, w1_ref, b1_ref, w2_ref, b2_ref, o_ref):
    h = jnp.dot(x_ref[...], w1_ref[...],
                preferred_element_type=jnp.float32) + b1_ref[...]
    logits = jnp.dot(h, w2_ref[...],
                     preferred_element_type=jnp.float32) + b2_ref[...]
    mx = jnp.max(logits, axis=-1, keepdims=True)
    e = jnp.exp(logits - mx)
    o_ref[...] = e / jnp.sum(e, axis=-1, keepdims=True)


def _fc_head(feat, w1, b1, w2, b2):
    n, f = feat.shape
    h = w1.shape[1]
    o = w2.shape[1]
    return pl.pallas_call(
        _fc_head_body,
        out_shape=jax.ShapeDtypeStruct((n, o), jnp.float32),
        grid=(1,),
        in_specs=[
            pl.BlockSpec((n, f), lambda i: (0, 0)),
            pl.BlockSpec((f, h), lambda i: (0, 0)),
            pl.BlockSpec((1, h), lambda i: (0, 0)),
            pl.BlockSpec((h, o), lambda i: (0, 0)),
            pl.BlockSpec((1, o), lambda i: (0, 0)),
        ],
        out_specs=pl.BlockSpec((n, o), lambda i: (0, 0)),
    )(feat, w1, b1, w2, b2)


def _pack0(x):
    n = x.shape[0]
    xb = jnp.pad(x.astype(_BF16), ((0, 0), (0, 0), (0, 1), (0, 17)))
    xb = xb.reshape(n, 6, 94, 4, 81, 4).transpose(0, 2, 4, 3, 5, 1)
    return xb.reshape(n, 94, 81, 96)


def _prep1(yv):
    xp = jnp.pad(yv.reshape(93, 80, 10), ((0, 3), (0, 52), (0, 0)))
    xp = xp.reshape(24, 4, 33, 4, 10).transpose(0, 2, 1, 3, 4)
    xp = xp.reshape(24, 33, 160)
    return [xp[:, :, q * 10:(q + 1) * 10] for q in range(16)]


def _prep2(yv):
    xp = jnp.pad(yv.reshape(23, 32, 20), ((0, 1), (0, 36), (0, 0)))
    xp = xp.reshape(6, 4, 17, 4, 20).transpose(0, 2, 1, 3, 4)
    xp = xp.reshape(6, 17, 320)
    return [xp[:, :, q * 20:(q + 1) * 20] for q in range(16)]


def kernel(x, w_mat0, shift0, w_mat1, shift1, w_mat2, shift2, w1, b1, w2, b2):
    n = x.shape[0]

    xp0 = _pack0(x)
    y0 = _conv_layer(
        [xp0], [pl.BlockSpec((None, 94, 81, 96), lambda i: (i, 0, 0, 0))],
        lambda refs: [refs[0][...][:, :, q * 6:(q + 1) * 6] for q in range(16)],
        w_mat0, shift0, 6, 93, 80, _BF16)
    y1 = _conv_layer(
        [y0], [pl.BlockSpec((None, 7440, 10), lambda i: (i, 0, 0))],
        lambda refs: _prep1(refs[0][...]),
        w_mat1, shift1, 10, 23, 32, _BF16)
    y2 = _conv_layer(
        [y1], [pl.BlockSpec((None, 736, 20), lambda i: (i, 0, 0))],
        lambda refs: _prep2(refs[0][...]),
        w_mat2, shift2, 20, 5, 16, jnp.float32)
    feat = y2.reshape(n, 5, 16, 40)[:, :, :4, :].reshape(n, 800)

    w1p = w1.reshape(40, 20, 100).transpose(1, 0, 2).reshape(800, 100)
    return _fc_head(feat, w1p, b1, w2, b2)


# speedup vs baseline: 17.5808x; 1.0005x over previous
import jax
import jax.numpy as jnp
from jax.experimental import pallas as pl
from jax.experimental.pallas import tpu as pltpu

_BF16 = jnp.bfloat16


def _make_conv_body(prep, c_in, rout, wd):
    def body(*args):
        x_refs, w_ref, b_ref, o_ref = args[:-3], args[-3], args[-2], args[-1]
        phases = prep(x_refs)
        w = w_ref[...]
        acc = None
        for dh in (0, 1):
            for dw in (0, 1):
                taps = []
                for kh in range(3):
                    for kw in range(3):
                        o = 2 * dh + kh
                        p = 2 * dw + kw
                        q = (o % 4) * 4 + (p % 4)
                        sh = o // 4
                        sw = p // 4
                        taps.append(
                            phases[q][sh:sh + rout, sw:sw + wd, :])
                patch = jnp.concatenate(taps, axis=-1)
                a = jnp.dot(patch.reshape(rout * wd, 9 * c_in), w,
                            preferred_element_type=jnp.float32)
                acc = a if acc is None else jnp.maximum(acc, a)
        o_ref[...] = jnp.maximum(acc + b_ref[...], 0.0).astype(o_ref.dtype)
    return body


def _conv_layer(xs, in_specs, prep, w_mat, shift, c_in, rout, wd, out_dtype):
    n = xs[0].shape[0]
    c_out = w_mat.shape[1]
    return pl.pallas_call(
        _make_conv_body(prep, c_in, rout, wd),
        out_shape=jax.ShapeDtypeStruct((n, rout * wd, c_out), out_dtype),
        grid=(n,),
        in_specs=in_specs + [
            pl.BlockSpec((9 * c_in, c_out), lambda i: (0, 0)),
            pl.BlockSpec((1, c_out), lambda i: (0, 0)),
        ],
        out_specs=pl.BlockSpec((None, rout * wd, c_out), lambda i: (i, 0, 0)),
        compiler_params=pltpu.CompilerParams(
            dimension_semantics=("parallel",)),
    )(*xs, w_mat.astype(_BF16), shift)


def _fc_head_body(x_ref, w1_ref, b1_ref, w2_ref, b2_ref, o_ref):
    h = jnp.dot(x_ref[...], w1_ref[...],
                preferred_element_type=jnp.float32) + b1_ref[...]
    logits = jnp.dot(h, w2_ref[...],
                     preferred_element_type=jnp.float32) + b2_ref[...]
    mx = jnp.max(logits, axis=-1, keepdims=True)
    e = jnp.exp(logits - mx)
    o_ref[...] = e / jnp.sum(e, axis=-1, keepdims=True)


def _fc_head(feat, w1, b1, w2, b2):
    n, f = feat.shape
    h = w1.shape[1]
    o = w2.shape[1]
    return pl.pallas_call(
        _fc_head_body,
        out_shape=jax.ShapeDtypeStruct((n, o), jnp.float32),
        grid=(1,),
        in_specs=[
            pl.BlockSpec((n, f), lambda i: (0, 0)),
            pl.BlockSpec((f, h), lambda i: (0, 0)),
            pl.BlockSpec((1, h), lambda i: (0, 0)),
            pl.BlockSpec((h, o), lambda i: (0, 0)),
            pl.BlockSpec((1, o), lambda i: (0, 0)),
        ],
        out_specs=pl.BlockSpec((n, o), lambda i: (0, 0)),
    )(feat, w1, b1, w2, b2)


def _pack0(x):
    n = x.shape[0]
    xb = jnp.pad(x, ((0, 0), (0, 0), (0, 1), (0, 17))).astype(_BF16)
    xb = xb.reshape(n, 6, 94, 4, 81, 4).transpose(0, 2, 4, 3, 5, 1)
    return xb.reshape(n, 94, 81, 96)


def _prep1(yv):
    xp = jnp.pad(yv.reshape(93, 80, 10), ((0, 3), (0, 52), (0, 0)))
    xp = xp.reshape(24, 4, 33, 4, 10).transpose(0, 2, 1, 3, 4)
    xp = xp.reshape(24, 33, 160)
    return [xp[:, :, q * 10:(q + 1) * 10] for q in range(16)]


def _prep2(yv):
    xp = jnp.pad(yv.reshape(23, 32, 20), ((0, 1), (0, 36), (0, 0)))
    xp = xp.reshape(6, 4, 17, 4, 20).transpose(0, 2, 1, 3, 4)
    xp = xp.reshape(6, 17, 320)
    return [xp[:, :, q * 20:(q + 1) * 20] for q in range(16)]


def kernel(x, w_mat0, shift0, w_mat1, shift1, w_mat2, shift2, w1, b1, w2, b2):
    n = x.shape[0]

    xp0 = _pack0(x)
    y0 = _conv_layer(
        [xp0], [pl.BlockSpec((None, 94, 81, 96), lambda i: (i, 0, 0, 0))],
        lambda refs: [refs[0][...][:, :, q * 6:(q + 1) * 6] for q in range(16)],
        w_mat0, shift0, 6, 93, 80, _BF16)
    y1 = _conv_layer(
        [y0], [pl.BlockSpec((None, 7440, 10), lambda i: (i, 0, 0))],
        lambda refs: _prep1(refs[0][...]),
        w_mat1, shift1, 10, 23, 32, _BF16)
    y2 = _conv_layer(
        [y1], [pl.BlockSpec((None, 736, 20), lambda i: (i, 0, 0))],
        lambda refs: _prep2(refs[0][...]),
        w_mat2, shift2, 20, 5, 16, jnp.float32)
    feat = y2.reshape(n, 5, 16, 40)[:, :, :4, :].reshape(n, 800)

    w1p = w1.reshape(40, 20, 100).transpose(1, 0, 2).reshape(800, 100)
    return _fc_head(feat, w1p, b1, w2, b2)
